# Initial kernel scaffold; baseline (speedup 1.0000x reference)
import jax
import jax.numpy as jnp
from jax.experimental import pallas as pl


def kernel(ids, table):
    raise NotImplementedError("write your pallas kernel here")


# speedup vs baseline: 1.0932x; 1.0932x over previous
import functools

import jax
import jax.numpy as jnp
from jax import lax
from jax.experimental import pallas as pl
from jax.experimental.pallas import tpu as pltpu
from jax.experimental.pallas import tpu_sc as plsc

_D = 32
_L = 16
_NC = 2
_NS = 16
_NW = _NC * _NS
_IPR = 128

_CHUNK = 1024
_K = _CHUNK // _IPR


def _gather_call(ids2d, table, *, b_total):
    rows_total = b_total // _IPR
    b_per_w = b_total // _NW
    n_chunk = b_per_w // _CHUNK
    rows_per_w = b_per_w // _IPR

    mesh = plsc.VectorSubcoreMesh(core_axis_name="c", subcore_axis_name="s")

    @functools.partial(
        pl.kernel,
        mesh=mesh,
        out_type=jax.ShapeDtypeStruct((b_total, _D), jnp.float32),
        compiler_params=pltpu.CompilerParams(use_tc_tiling_on_sc=False),
        scratch_types=[
            pltpu.VMEM((_K, _IPR), jnp.int32),
            pltpu.VMEM((_CHUNK, _D), jnp.float32),
            pltpu.SemaphoreType.DMA,
        ],
    )
    def k(ids_hbm, table_hbm, out_hbm, idx_v, rows_v, sem):
        wid = lax.axis_index("s") * _NC + lax.axis_index("c")
        row0 = wid * rows_per_w

        def chunk_body(c, carry):
            r = row0 + c * _K
            pltpu.sync_copy(ids_hbm.at[pl.ds(r, _K)], idx_v)
            for j in range(_K):
                for i in range(_IPR // _L):
                    sl = pl.ds(i * _L, _L)
                    idx_v[j, sl] = idx_v[j, sl] + 1
            copies = [
                pltpu.async_copy(
                    table_hbm.at[idx_v.at[j]],
                    rows_v.at[pl.ds(j * _IPR, _IPR)],
                    sem,
                )
                for j in range(_K)
            ]
            for cp in copies:
                cp.wait()
            pltpu.sync_copy(rows_v, out_hbm.at[pl.ds(r * _IPR, _CHUNK)])
            return carry

        lax.fori_loop(0, n_chunk, chunk_body, 0)

    return k(ids2d, table)


def kernel(ids, table):
    b, h = ids.shape
    b_total = b * h
    ids2d = ids.reshape(b_total // _IPR, _IPR)
    out = _gather_call(ids2d, table, b_total=b_total)
    return out.reshape(b, h, _D)


# speedup vs baseline: 1.1090x; 1.0145x over previous
import functools

import jax
import jax.numpy as jnp
from jax import lax
from jax.experimental import pallas as pl
from jax.experimental.pallas import tpu as pltpu
from jax.experimental.pallas import tpu_sc as plsc

_D = 32
_L = 16
_NC = 2
_NS = 16
_NW = _NC * _NS
_IPR = 128

_CHUNK = 1280
_K = _CHUNK // _IPR


def _gather_call(ids2d, table, *, b_total):
    b_per_w = b_total // _NW
    n_chunk = b_per_w // _CHUNK
    rows_per_w = b_per_w // _IPR
    assert n_chunk >= 4 and n_chunk % 2 == 0

    mesh = plsc.VectorSubcoreMesh(core_axis_name="c", subcore_axis_name="s")

    @functools.partial(
        pl.kernel,
        mesh=mesh,
        out_type=jax.ShapeDtypeStruct((b_total, _D), jnp.float32),
        compiler_params=pltpu.CompilerParams(use_tc_tiling_on_sc=False),
        scratch_types=[
            pltpu.VMEM((2, _K, _IPR), jnp.int32),
            pltpu.VMEM((2, _CHUNK, _D), jnp.float32),
            pltpu.SemaphoreType.DMA,
            pltpu.SemaphoreType.DMA,
            pltpu.SemaphoreType.DMA,
            pltpu.SemaphoreType.DMA,
            pltpu.SemaphoreType.DMA,
            pltpu.SemaphoreType.DMA,
        ],
    )
    def k(ids_hbm, table_hbm, out_hbm, idx_v, rows_v,
          sem_i0, sem_i1, sem_g0, sem_g1, sem_o0, sem_o1):
        wid = lax.axis_index("s") * _NC + lax.axis_index("c")
        row0 = wid * rows_per_w
        sem_i = (sem_i0, sem_i1)
        sem_g = (sem_g0, sem_g1)
        sem_o = (sem_o0, sem_o1)

        def idx_cp(c, slot):
            return pltpu.make_async_copy(
                ids_hbm.at[pl.ds(row0 + c * _K, _K)], idx_v.at[slot],
                sem_i[slot])

        def gather_cp(j, slot):
            return pltpu.make_async_copy(
                table_hbm.at[idx_v.at[slot, j]],
                rows_v.at[slot, pl.ds(j * _IPR, _IPR)], sem_g[slot])

        def out_cp(c, slot):
            return pltpu.make_async_copy(
                rows_v.at[slot],
                out_hbm.at[pl.ds((row0 + c * _K) * _IPR, _CHUNK)],
                sem_o[slot])

        def plusone(slot):
            for j in range(_K):
                for t in range(_IPR // _L):
                    sl = pl.ds(t * _L, _L)
                    idx_v[slot, j, sl] = idx_v[slot, j, sl] + 1

        def fire_gathers(slot):
            for j in range(_K):
                gather_cp(j, slot).start()

        def drain_gathers(slot):
            for j in range(_K):
                gather_cp(j, slot).wait()

        def step(c, slot, *, first=False, prefetch=True, has_next=True):
            if has_next:
                idx_cp(0, 1 - slot).wait()
                plusone(1 - slot)
            drain_gathers(slot)
            if prefetch:
                idx_cp(c + 2, slot).start()
            if has_next:
                if not first:
                    out_cp(0, 1 - slot).wait()
                fire_gathers(1 - slot)
            out_cp(c, slot).start()

        idx_cp(0, 0).start()
        idx_cp(1, 1).start()
        idx_cp(0, 0).wait()
        plusone(0)
        fire_gathers(0)

        step(0, 0, first=True)
        step(1, 1)

        def body(i, carry):
            step(2 * i, 0)
            step(2 * i + 1, 1)
            return carry

        lax.fori_loop(1, n_chunk // 2 - 1, body, 0)

        step(n_chunk - 2, 0, prefetch=False)
        step(n_chunk - 1, 1, prefetch=False, has_next=False)
        out_cp(0, 0).wait()
        out_cp(0, 1).wait()

    return k(ids2d, table)


def kernel(ids, table):
    b, h = ids.shape
    b_total = b * h
    ids2d = ids.reshape(b_total // _IPR, _IPR)
    out = _gather_call(ids2d, table, b_total=b_total)
    return out.reshape(b, h, _D)


# speedup vs baseline: 1.1102x; 1.0011x over previous
import functools

import jax
import jax.numpy as jnp
from jax import lax
from jax.experimental import pallas as pl
from jax.experimental.pallas import tpu as pltpu
from jax.experimental.pallas import tpu_sc as plsc

_D = 32
_L = 16
_NC = 2
_NS = 16
_NW = _NC * _NS

_CHUNK = 1280


def _gather_call(ids1d, table, *, b_total):
    b_per_w = b_total // _NW
    n_chunk = b_per_w // _CHUNK
    assert n_chunk >= 4 and n_chunk % 2 == 0

    mesh = plsc.VectorSubcoreMesh(core_axis_name="c", subcore_axis_name="s")

    @functools.partial(
        pl.kernel,
        mesh=mesh,
        out_type=jax.ShapeDtypeStruct((b_total, _D), jnp.float32),
        compiler_params=pltpu.CompilerParams(use_tc_tiling_on_sc=False),
        scratch_types=[
            pltpu.VMEM((2, _CHUNK), jnp.int32),
            pltpu.VMEM((2, _CHUNK, _D), jnp.float32),
            pltpu.SemaphoreType.DMA,
            pltpu.SemaphoreType.DMA,
            pltpu.SemaphoreType.DMA,
            pltpu.SemaphoreType.DMA,
            pltpu.SemaphoreType.DMA,
            pltpu.SemaphoreType.DMA,
        ],
    )
    def k(ids_hbm, table_hbm, out_hbm, idx_v, rows_v,
          sem_i0, sem_i1, sem_g0, sem_g1, sem_o0, sem_o1):
        wid = lax.axis_index("s") * _NC + lax.axis_index("c")
        base = wid * b_per_w
        sem_i = (sem_i0, sem_i1)
        sem_g = (sem_g0, sem_g1)
        sem_o = (sem_o0, sem_o1)

        def idx_cp(c, slot):
            return pltpu.make_async_copy(
                ids_hbm.at[pl.ds(base + c * _CHUNK, _CHUNK)],
                idx_v.at[slot], sem_i[slot])

        def gather_cp(slot):
            return pltpu.make_async_copy(
                table_hbm.at[idx_v.at[slot]], rows_v.at[slot], sem_g[slot])

        def out_cp(c, slot):
            return pltpu.make_async_copy(
                rows_v.at[slot],
                out_hbm.at[pl.ds(base + c * _CHUNK, _CHUNK)],
                sem_o[slot])

        def plusone(slot):
            for t in range(_CHUNK // _L):
                sl = pl.ds(t * _L, _L)
                idx_v[slot, sl] = idx_v[slot, sl] + 1

        def step(c, slot, *, first=False, prefetch=True, has_next=True):
            if has_next:
                idx_cp(0, 1 - slot).wait()
                plusone(1 - slot)
            gather_cp(slot).wait()
            if prefetch:
                idx_cp(c + 2, slot).start()
            if has_next:
                if not first:
                    out_cp(0, 1 - slot).wait()
                gather_cp(1 - slot).start()
            out_cp(c, slot).start()

        idx_cp(0, 0).start()
        idx_cp(1, 1).start()
        idx_cp(0, 0).wait()
        plusone(0)
        gather_cp(0).start()

        step(0, 0, first=True)
        step(1, 1)

        def body(i, carry):
            step(2 * i, 0)
            step(2 * i + 1, 1)
            return carry

        lax.fori_loop(1, n_chunk // 2 - 1, body, 0)

        step(n_chunk - 2, 0, prefetch=False)
        step(n_chunk - 1, 1, prefetch=False, has_next=False)
        out_cp(0, 0).wait()
        out_cp(0, 1).wait()

    return k(ids1d, table)


def kernel(ids, table):
    b, h = ids.shape
    b_total = b * h
    ids1d = ids.reshape(b_total)
    out = _gather_call(ids1d, table, b_total=b_total)
    return out.reshape(b, h, _D)


# speedup vs baseline: 1.5051x; 1.3556x over previous
import functools

import jax
import jax.numpy as jnp
from jax import lax
from jax.experimental import pallas as pl
from jax.experimental.pallas import tpu as pltpu
from jax.experimental.pallas import tpu_sc as plsc

_D = 32
_L = 16
_NC = 2
_NS = 16
_NW = _NC * _NS


def _gather_call(ids_t, table, *, batch, hist):
    nb = batch // _NW
    assert hist % 2 == 0 and hist >= 6 and nb % _L == 0

    mesh = plsc.VectorSubcoreMesh(core_axis_name="c", subcore_axis_name="s")

    @functools.partial(
        pl.kernel,
        mesh=mesh,
        out_type=jax.ShapeDtypeStruct((hist, _D, batch), jnp.float32),
        compiler_params=pltpu.CompilerParams(
            use_tc_tiling_on_sc=False, needs_layout_passes=False),
        scratch_types=[
            pltpu.VMEM((2, nb), jnp.int32),
            pltpu.VMEM((2, nb, _D), jnp.float32),
            pltpu.VMEM((2, _D, nb), jnp.float32),
            pltpu.SemaphoreType.DMA,
            pltpu.SemaphoreType.DMA,
            pltpu.SemaphoreType.DMA,
            pltpu.SemaphoreType.DMA,
            pltpu.SemaphoreType.DMA,
            pltpu.SemaphoreType.DMA,
        ],
    )
    def k(ids_hbm, table_hbm, out_hbm, idx_v, rows_v, trans_v,
          sem_i0, sem_i1, sem_g0, sem_g1, sem_o0, sem_o1):
        wid = lax.axis_index("s") * _NC + lax.axis_index("c")
        b0 = wid * nb
        sem_i = (sem_i0, sem_i1)
        sem_g = (sem_g0, sem_g1)
        sem_o = (sem_o0, sem_o1)

        def idx_cp(h, slot):
            return pltpu.make_async_copy(
                ids_hbm.at[h, pl.ds(b0, nb)], idx_v.at[slot], sem_i[slot])

        def gather_cp(slot):
            return pltpu.make_async_copy(
                table_hbm.at[idx_v.at[slot]], rows_v.at[slot], sem_g[slot])

        def out_cp(h, slot):
            return pltpu.make_async_copy(
                trans_v.at[slot],
                out_hbm.at[h, :, pl.ds(b0, nb)], sem_o[slot])

        def plusone(slot):
            for t in range(nb // _L):
                sl = pl.ds(t * _L, _L)
                idx_v[slot, sl] = idx_v[slot, sl] + 1

        lanes = lax.iota(jnp.int32, _L)

        def transpose(slot):
            rows = rows_v.at[slot]

            def tbody(g, carry):
                bidx = lanes + g * _L
                off = g * _L
                for d in range(_D):
                    didx = jnp.full((_L,), d, jnp.int32)
                    vals = plsc.load_gather(rows, [bidx, didx])
                    trans_v[slot, d, pl.ds(off, _L)] = vals
                return carry

            lax.fori_loop(0, nb // _L, tbody, 0)

        def step(h, slot, *, warmup=False, prefetch=True, has_next=True):
            if has_next:
                idx_cp(0, 1 - slot).wait()
                plusone(1 - slot)
            gather_cp(slot).wait()
            if prefetch:
                idx_cp(h + 2, slot).start()
            if has_next:
                gather_cp(1 - slot).start()
            if not warmup:
                out_cp(0, slot).wait()
            transpose(slot)
            out_cp(h, slot).start()

        idx_cp(0, 0).start()
        idx_cp(1, 1).start()
        idx_cp(0, 0).wait()
        plusone(0)
        gather_cp(0).start()

        step(0, 0, warmup=True)
        step(1, 1, warmup=True)

        def body(i, carry):
            step(2 * i, 0)
            step(2 * i + 1, 1)
            return carry

        lax.fori_loop(1, hist // 2 - 1, body, 0)

        step(hist - 2, 0, prefetch=False)
        step(hist - 1, 1, prefetch=False, has_next=False)
        out_cp(0, 0).wait()
        out_cp(0, 1).wait()

    return k(ids_t, table)


def kernel(ids, table):
    b, h = ids.shape
    ids_t = ids.T
    out_t = _gather_call(ids_t, table, batch=b, hist=h)
    return jnp.transpose(out_t, (2, 0, 1))
